# baseline (speedup 1.0000x reference)
import functools

import jax
import jax.numpy as jnp
from jax import lax
from jax.experimental import pallas as pl
from jax.experimental.pallas import tpu as pltpu
from jax.experimental.pallas import tpu_sc as plsc

N = 10000
D = 128
H = 50
C = 10
K = 5

HP = 64
HH = 32
NPAD = 10240
RPT = NPAD // 16
NW = 32
CE = 128
BN = 1024

NBUF = 8
GAHEAD = 4

KC0 = 80
KC1 = 80
KMAX = max(KC0, KC1)
TOTCH = 16 * (KC0 + KC1)
EPAD = TOTCH * CE
DCH = TOTCH // NW

_sc_mesh = plsc.VectorSubcoreMesh(core_axis_name="c", subcore_axis_name="s")


@functools.partial(
    pl.kernel,
    mesh=_sc_mesh,
    compiler_params=pltpu.CompilerParams(use_tc_tiling_on_sc=False),
    out_type=jax.ShapeDtypeStruct((2, NPAD, 16), jnp.float32),
    scratch_types=[
        pltpu.VMEM((DCH, CE), jnp.int32),
        pltpu.VMEM((CE, 16), jnp.float32),
        pltpu.VMEM((CE, 16), jnp.float32),
        pltpu.VMEM_SHARED((NPAD, 16), jnp.float32),
    ],
)
def _deg_kernel(ridx_hbm, part_hbm, idx_v, ones_v, zrow_v, acc_sh):
    c = lax.axis_index("c")
    s = lax.axis_index("s")
    wid = c * 16 + s
    pltpu.sync_copy(ridx_hbm.at[pl.ds(wid * DCH, DCH)], idx_v)

    lane = lax.iota(jnp.int32, 16)
    onehot = jnp.where(lane == 0, 1.0, 0.0).astype(jnp.float32)
    zero16 = jnp.zeros((16,), jnp.float32)

    def fill(i, _):
        ones_v[i, :] = onehot
        zrow_v[i, :] = zero16
        return 0

    lax.fori_loop(0, CE, fill, 0)

    def zcp(i, _):
        pltpu.sync_copy(zrow_v, acc_sh.at[pl.ds(s * RPT + i * CE, CE)])
        return 0

    lax.fori_loop(0, RPT // CE, zcp, 0)
    plsc.subcore_barrier()

    def scat(j, _):
        pltpu.sync_copy(ones_v, acc_sh.at[idx_v.at[j]], add=True)
        return 0

    lax.fori_loop(0, DCH, scat, 0)
    plsc.subcore_barrier()
    pltpu.sync_copy(acc_sh.at[pl.ds(s * RPT, RPT)],
                    part_hbm.at[c, pl.ds(s * RPT, RPT)])


@functools.partial(
    pl.kernel,
    mesh=_sc_mesh,
    compiler_params=pltpu.CompilerParams(use_tc_tiling_on_sc=False),
    out_type=jax.ShapeDtypeStruct((2, 2, NPAD, HH), jnp.float32),
    scratch_types=[
        pltpu.VMEM((KMAX, CE), jnp.int32),
        pltpu.VMEM((KMAX, CE), jnp.int32),
        pltpu.VMEM((CE, HH), jnp.float32),
        [pltpu.VMEM((CE, HH), jnp.float32)] * NBUF,
        pltpu.VMEM_SHARED((NPAD, HH), jnp.float32),
        pltpu.VMEM_SHARED((NPAD, HH), jnp.float32),
        pltpu.VMEM_SHARED((NPAD, HH), jnp.float32),
        [pltpu.SemaphoreType.DMA] * NBUF,
        [pltpu.SemaphoreType.DMA] * NBUF,
        pltpu.SemaphoreType.REGULAR,
    ],
)
def _mega_kernel(w0_hbm, ridx_hbm, cidx_hbm, qs_hbm, dds_hbm, part_hbm,
                 ridx_v, cidx_v, zbuf, rows, acc_sh, wsh0, wsh1, sg, ss,
                 bsem):
    c = lax.axis_index("c")
    s = lax.axis_index("s")
    myk = jnp.where(c == 0, KC0, KC1)
    myoff = jnp.where(c == 0, s * KC0, 16 * KC0 + s * KC1)
    pltpu.sync_copy(ridx_hbm.at[pl.ds(myoff, KMAX)], ridx_v)
    pltpu.sync_copy(cidx_hbm.at[pl.ds(myoff, KMAX)], cidx_v)

    zero16 = jnp.zeros((16,), jnp.float32)

    def zfill(i, _):
        zbuf[i // 2, pl.ds((i % 2) * 16, 16)] = zero16
        return 0

    lax.fori_loop(0, CE * (HH // 16), zfill, 0)

    pltpu.sync_copy(w0_hbm.at[0, pl.ds(s * RPT, RPT)],
                    wsh0.at[pl.ds(s * RPT, RPT)])
    pltpu.sync_copy(w0_hbm.at[1, pl.ds(s * RPT, RPT)],
                    wsh1.at[pl.ds(s * RPT, RPT)])

    for step in range(4):
        for h, wsh in ((0, wsh0), (1, wsh1)):
            def zcp(i, _):
                pltpu.sync_copy(zbuf, acc_sh.at[pl.ds(s * RPT + i * CE, CE)])
                return 0

            lax.fori_loop(0, RPT // CE, zcp, 0)
            plsc.subcore_barrier()
            for b in range(GAHEAD):
                pltpu.async_copy(wsh.at[ridx_v.at[b]], rows[b], sg[b])

            def rnd(r, _):
                for b in range(NBUF):
                    j = r * NBUF + b
                    b4 = (b + GAHEAD) % NBUF
                    pltpu.make_async_copy(wsh.at[ridx_v.at[0]], rows[b],
                                          sg[b]).wait()
                    pltpu.async_copy(rows[b], acc_sh.at[cidx_v.at[j]],
                                     ss[b], add=True)

                    @pl.when(j >= GAHEAD)
                    def _():
                        pltpu.make_async_copy(rows[b4],
                                              acc_sh.at[cidx_v.at[0]],
                                              ss[b4]).wait()

                    @pl.when(j + GAHEAD < myk)
                    def _():
                        pltpu.async_copy(wsh.at[ridx_v.at[j + GAHEAD]],
                                         rows[b4], sg[b4])
                return 0

            lax.fori_loop(0, myk // NBUF, rnd, 0)
            for b in range(GAHEAD, NBUF):
                pltpu.make_async_copy(rows[b], acc_sh.at[cidx_v.at[0]],
                                      ss[b]).wait()
            plsc.subcore_barrier()
            pltpu.sync_copy(acc_sh.at[pl.ds(s * RPT, RPT)],
                            part_hbm.at[c, h, pl.ds(s * RPT, RPT)])

        if step == 3:
            break

        plsc.subcore_barrier()
        pltpu.core_barrier(bsem, core_axis_name="c")
        for h, wsh in ((0, wsh0), (1, wsh1)):
            for blk in range(RPT // CE):
                base = s * RPT + blk * CE
                pltpu.sync_copy(part_hbm.at[0, h, pl.ds(base, CE)], rows[0])
                pltpu.sync_copy(part_hbm.at[1, h, pl.ds(base, CE)], rows[1])
                pltpu.sync_copy(qs_hbm.at[step, h, pl.ds(base, CE)], rows[2])
                pltpu.sync_copy(dds_hbm.at[pl.ds(base, CE)], rows[3])

                def comb(i, _):
                    r = i // 2
                    l0 = (i % 2) * 16
                    srow = rows[0][r, pl.ds(l0, 16)] + rows[1][r, pl.ds(l0, 16)]
                    rows[4][r, pl.ds(l0, 16)] = (
                        rows[2][r, pl.ds(l0, 16)]
                        - rows[3][r, pl.ds(l0, 16)] * srow)
                    return 0

                lax.fori_loop(0, CE * (HH // 16), comb, 0)
                pltpu.sync_copy(rows[4], wsh.at[pl.ds(base, CE)])
        plsc.subcore_barrier()
        pltpu.core_barrier(bsem, core_axis_name="c")


def _mm1_body(x_ref, w_ref, degp_ref, qp_ref, w0_ref, qs_ref, dis_ref, dds_ref):
    deg = degp_ref[0, :, 0:1] + degp_ref[1, :, 0:1]
    dis = jnp.where(deg > 0, lax.rsqrt(jnp.maximum(deg, 1e-12)), 0.0)
    p = jnp.dot(x_ref[...], w_ref[...], preferred_element_type=jnp.float32)
    for k in range(4):
        qp_ref[k] = p[:, k * HP:(k + 1) * HP] * dis
    qp_ref[4] = p[:, 4 * HP:5 * HP]
    q4 = p[:, 0:HP] * dis
    w0_ref[0] = q4[:, :HH]
    w0_ref[1] = q4[:, HH:]
    for kk, k in enumerate((1, 2, 3)):
        qk = p[:, k * HP:(k + 1) * HP] * dis
        qs_ref[kk, 0] = qk[:, :HH]
        qs_ref[kk, 1] = qk[:, HH:]
    dis_ref[...] = jnp.broadcast_to(dis, (BN, HP))
    dds_ref[...] = jnp.broadcast_to(dis * dis, (BN, HH))


def _mm1(x_p, ccat, degp):
    return pl.pallas_call(
        _mm1_body,
        grid=(NPAD // BN,),
        in_specs=[
            pl.BlockSpec((BN, D), lambda i: (i, 0)),
            pl.BlockSpec((D, 5 * HP), lambda i: (0, 0)),
            pl.BlockSpec((2, BN, 16), lambda i: (0, i, 0)),
        ],
        out_specs=[
            pl.BlockSpec((5, BN, HP), lambda i: (0, i, 0)),
            pl.BlockSpec((2, BN, HH), lambda i: (0, i, 0)),
            pl.BlockSpec((3, 2, BN, HH), lambda i: (0, 0, i, 0)),
            pl.BlockSpec((BN, HP), lambda i: (i, 0)),
            pl.BlockSpec((BN, HH), lambda i: (i, 0)),
        ],
        out_shape=[
            jax.ShapeDtypeStruct((5, NPAD, HP), jnp.float32),
            jax.ShapeDtypeStruct((2, NPAD, HH), jnp.float32),
            jax.ShapeDtypeStruct((3, 2, NPAD, HH), jnp.float32),
            jax.ShapeDtypeStruct((NPAD, HP), jnp.float32),
            jax.ShapeDtypeStruct((NPAD, HH), jnp.float32),
        ],
    )(x_p, ccat, degp)


def _final_body(p_ref, dis_ref, p0_ref, bc_ref, wf_ref, bf_ref, o_ref):
    srow = jnp.concatenate(
        [p_ref[0, 0] + p_ref[1, 0], p_ref[0, 1] + p_ref[1, 1]], axis=1)
    pre = p0_ref[...] - dis_ref[...] * srow + bc_ref[...]
    h = jnp.maximum(pre, 0.0)
    logits = jnp.dot(h, wf_ref[...], preferred_element_type=jnp.float32)
    logits = logits + bf_ref[...]
    m = jnp.max(logits, axis=1, keepdims=True)
    lse = jnp.log(jnp.sum(jnp.exp(logits - m), axis=1, keepdims=True)) + m
    o_ref[...] = logits - lse


def _final(part, dis2d, p0, bc_p, wf_p, bf_p):
    return pl.pallas_call(
        _final_body,
        grid=(NPAD // BN,),
        in_specs=[
            pl.BlockSpec((2, 2, BN, HH), lambda i: (0, 0, i, 0)),
            pl.BlockSpec((BN, HP), lambda i: (i, 0)),
            pl.BlockSpec((BN, HP), lambda i: (i, 0)),
            pl.BlockSpec((1, HP), lambda i: (0, 0)),
            pl.BlockSpec((HP, C), lambda i: (0, 0)),
            pl.BlockSpec((1, C), lambda i: (0, 0)),
        ],
        out_specs=pl.BlockSpec((BN, C), lambda i: (i, 0)),
        out_shape=jax.ShapeDtypeStruct((NPAD, C), jnp.float32),
    )(part, dis2d, p0, bc_p, wf_p, bf_p)


def kernel(x, edge_index, W_cheb, b_cheb, W_fc, b_fc):
    c0 = W_cheb[0] - W_cheb[2] + W_cheb[4]
    c1 = W_cheb[1] - 3.0 * W_cheb[3]
    c2 = 2.0 * W_cheb[2] - 8.0 * W_cheb[4]
    c3 = 4.0 * W_cheb[3]
    c4 = 8.0 * W_cheb[4]
    pad = [(0, 0), (0, HP - H)]
    ccat = jnp.concatenate(
        [jnp.pad(m, pad) for m in (c4, c3, c2, c1, c0)], axis=1)

    x_p = jnp.pad(x, [(0, NPAD - N), (0, 0)])
    row_p = jnp.pad(edge_index[0], (0, EPAD - edge_index.shape[1]),
                    constant_values=N).reshape(TOTCH, CE)
    col_p = jnp.pad(edge_index[1], (0, EPAD - edge_index.shape[1]),
                    constant_values=N).reshape(TOTCH, CE)

    bc_p = jnp.pad(b_cheb, (0, HP - H)).reshape(1, HP)
    wf_p = jnp.pad(W_fc, [(0, HP - H), (0, 0)])
    bf_p = b_fc.reshape(1, C)

    degp = _deg_kernel(row_p)
    qp5, w0, qs, dis2d, dds = _mm1(x_p, ccat, degp)
    part = _mega_kernel(w0, row_p, col_p, qs, dds)
    out = _final(part, dis2d, qp5[4], bc_p, wf_p, bf_p)
    return out[:N]


# speedup vs baseline: 1.1782x; 1.1782x over previous
import functools

import jax
import jax.numpy as jnp
from jax import lax
from jax.experimental import pallas as pl
from jax.experimental.pallas import tpu as pltpu
from jax.experimental.pallas import tpu_sc as plsc

N = 10000
D = 128
H = 50
C = 10
K = 5

HP = 64
HH = 32
NPAD = 10240
RPT = NPAD // 16
NW = 32
CE = 128
BN = 1024

NBUF = 8
GAHEAD = 4

KC0 = 80
KC1 = 80
KMAX = max(KC0, KC1)
TOTCH = 16 * (KC0 + KC1)
EPAD = TOTCH * CE
DCH = TOTCH // NW

_sc_mesh = plsc.VectorSubcoreMesh(core_axis_name="c", subcore_axis_name="s")


@functools.partial(
    pl.kernel,
    mesh=_sc_mesh,
    compiler_params=pltpu.CompilerParams(use_tc_tiling_on_sc=False),
    out_type=jax.ShapeDtypeStruct((2, NPAD, 16), jnp.float32),
    scratch_types=[
        pltpu.VMEM((DCH, CE), jnp.int32),
        pltpu.VMEM((CE, 16), jnp.float32),
        pltpu.VMEM((CE, 16), jnp.float32),
        pltpu.VMEM_SHARED((NPAD, 16), jnp.float32),
    ],
)
def _deg_kernel(ridx_hbm, part_hbm, idx_v, ones_v, zrow_v, acc_sh):
    c = lax.axis_index("c")
    s = lax.axis_index("s")
    wid = c * 16 + s
    pltpu.sync_copy(ridx_hbm.at[pl.ds(wid * DCH, DCH)], idx_v)

    lane = lax.iota(jnp.int32, 16)
    onehot = jnp.where(lane == 0, 1.0, 0.0).astype(jnp.float32)
    zero16 = jnp.zeros((16,), jnp.float32)

    def fill(i, _):
        ones_v[i, :] = onehot
        zrow_v[i, :] = zero16
        return 0

    lax.fori_loop(0, CE, fill, 0)

    def zcp(i, _):
        pltpu.sync_copy(zrow_v, acc_sh.at[pl.ds(s * RPT + i * CE, CE)])
        return 0

    lax.fori_loop(0, RPT // CE, zcp, 0)
    plsc.subcore_barrier()

    def scat(j, _):
        pltpu.sync_copy(ones_v, acc_sh.at[idx_v.at[j]], add=True)
        return 0

    lax.fori_loop(0, DCH, scat, 0)
    plsc.subcore_barrier()
    pltpu.sync_copy(acc_sh.at[pl.ds(s * RPT, RPT)],
                    part_hbm.at[c, pl.ds(s * RPT, RPT)])


@functools.partial(
    pl.kernel,
    mesh=_sc_mesh,
    compiler_params=pltpu.CompilerParams(use_tc_tiling_on_sc=False),
    out_type=jax.ShapeDtypeStruct((2, 2, NPAD, HH), jnp.float32),
    scratch_types=[
        pltpu.VMEM((KMAX, CE), jnp.int32),
        pltpu.VMEM((KMAX, CE), jnp.int32),
        pltpu.VMEM((CE, HH), jnp.float32),
        [pltpu.VMEM((CE, HH), jnp.float32)] * NBUF,
        pltpu.VMEM_SHARED((NPAD, HH), jnp.float32),
        pltpu.VMEM_SHARED((NPAD, HH), jnp.float32),
        pltpu.VMEM_SHARED((NPAD, HH), jnp.float32),
        [pltpu.SemaphoreType.DMA] * NBUF,
        [pltpu.SemaphoreType.DMA] * NBUF,
        pltpu.SemaphoreType.REGULAR,
    ],
)
def _mega_kernel(w0_hbm, ridx_hbm, cidx_hbm, qs_hbm, dds_hbm, part_hbm,
                 ridx_v, cidx_v, zbuf, rows, acc_sh, wsh0, wsh1,
                 sg, ss, bsem):
    c = lax.axis_index("c")
    s = lax.axis_index("s")
    myk = jnp.where(c == 0, KC0, KC1)
    myoff = jnp.where(c == 0, s * KC0, 16 * KC0 + s * KC1)
    pltpu.sync_copy(ridx_hbm.at[pl.ds(myoff, KMAX)], ridx_v)
    pltpu.sync_copy(cidx_hbm.at[pl.ds(myoff, KMAX)], cidx_v)

    zero16 = jnp.zeros((16,), jnp.float32)

    def zfill(i, _):
        zbuf[i // 2, pl.ds((i % 2) * 16, 16)] = zero16
        return 0

    lax.fori_loop(0, CE * (HH // 16), zfill, 0)

    pltpu.sync_copy(w0_hbm.at[0, pl.ds(s * RPT, RPT)],
                    wsh0.at[pl.ds(s * RPT, RPT)])
    pltpu.sync_copy(w0_hbm.at[1, pl.ds(s * RPT, RPT)],
                    wsh1.at[pl.ds(s * RPT, RPT)])

    for step in range(4):
        for h, wsh in ((0, wsh0), (1, wsh1)):
            def zcp(i, _):
                pltpu.sync_copy(zbuf, acc_sh.at[pl.ds(s * RPT + i * CE, CE)])
                return 0

            lax.fori_loop(0, RPT // CE, zcp, 0)
            plsc.subcore_barrier()
            for b in range(GAHEAD):
                pltpu.async_copy(wsh.at[ridx_v.at[b]], rows[b], sg[b])

            def rnd(r, _):
                for b in range(NBUF):
                    j = r * NBUF + b
                    b4 = (b + GAHEAD) % NBUF
                    pltpu.make_async_copy(wsh.at[ridx_v.at[0]], rows[b],
                                          sg[b]).wait()
                    pltpu.async_copy(rows[b], acc_sh.at[cidx_v.at[j]],
                                     ss[b], add=True)

                    @pl.when(j >= GAHEAD)
                    def _():
                        pltpu.make_async_copy(rows[b4],
                                              acc_sh.at[cidx_v.at[0]],
                                              ss[b4]).wait()

                    @pl.when(j + GAHEAD < myk)
                    def _():
                        pltpu.async_copy(wsh.at[ridx_v.at[j + GAHEAD]],
                                         rows[b4], sg[b4])
                return 0

            lax.fori_loop(0, myk // NBUF, rnd, 0)
            for b in range(GAHEAD, NBUF):
                pltpu.make_async_copy(rows[b], acc_sh.at[cidx_v.at[0]],
                                      ss[b]).wait()
            plsc.subcore_barrier()
            if step == 3:
                pltpu.core_barrier(bsem, core_axis_name="c")
                pltpu.sync_copy(acc_sh.at[pl.ds(s * RPT, RPT)],
                                part_hbm.at[c, h, pl.ds(s * RPT, RPT)])
                continue
            pltpu.sync_copy(acc_sh.at[pl.ds(s * RPT, RPT)],
                            part_hbm.at[c, h, pl.ds(s * RPT, RPT)])

            pltpu.core_barrier(bsem, core_axis_name="c")

            def oth_src(blk):
                return part_hbm.at[1 - c, h, pl.ds(s * RPT + blk * CE, CE)]

            def q_src(blk):
                return qs_hbm.at[step, h, pl.ds(s * RPT + blk * CE, CE)]

            def dd_src(blk):
                return dds_hbm.at[pl.ds(s * RPT + blk * CE, CE)]

            nblk = RPT // CE
            pltpu.async_copy(oth_src(0), rows[0], sg[0])
            pltpu.async_copy(q_src(0), rows[1], sg[1])
            pltpu.async_copy(dd_src(0), rows[2], sg[2])
            for t in range(nblk):
                e = (t % 2) * 3
                o = ((t + 1) % 2) * 3
                pltpu.make_async_copy(oth_src(t), rows[e], sg[e]).wait()
                pltpu.make_async_copy(q_src(t), rows[e + 1],
                                      sg[e + 1]).wait()
                pltpu.make_async_copy(dd_src(t), rows[e + 2],
                                      sg[e + 2]).wait()
                if t + 1 < nblk:
                    pltpu.async_copy(oth_src(t + 1), rows[o], sg[o])
                    pltpu.async_copy(q_src(t + 1), rows[o + 1], sg[o + 1])
                    pltpu.async_copy(dd_src(t + 1), rows[o + 2], sg[o + 2])
                pltpu.sync_copy(acc_sh.at[pl.ds(s * RPT + t * CE, CE)],
                                rows[6])

                def comb(i, _):
                    for u in range(4):
                        ii = i * 4 + u
                        r = ii // 2
                        l0 = (ii % 2) * 16
                        srow = (rows[6][r, pl.ds(l0, 16)]
                                + rows[e][r, pl.ds(l0, 16)])
                        rows[7][r, pl.ds(l0, 16)] = (
                            rows[e + 1][r, pl.ds(l0, 16)]
                            - rows[e + 2][r, pl.ds(l0, 16)] * srow)
                    return 0

                lax.fori_loop(0, CE * (HH // 16) // 4, comb, 0)
                pltpu.sync_copy(rows[7],
                                wsh.at[pl.ds(s * RPT + t * CE, CE)])


def _mm1_body(x_ref, w_ref, degp_ref, qp_ref, w0_ref, qs_ref, dis_ref, dds_ref):
    deg = degp_ref[0, :, 0:1] + degp_ref[1, :, 0:1]
    dis = jnp.where(deg > 0, lax.rsqrt(jnp.maximum(deg, 1e-12)), 0.0)
    p = jnp.dot(x_ref[...], w_ref[...], preferred_element_type=jnp.float32)
    for k in range(4):
        qp_ref[k] = p[:, k * HP:(k + 1) * HP] * dis
    qp_ref[4] = p[:, 4 * HP:5 * HP]
    q4 = p[:, 0:HP] * dis
    w0_ref[0] = q4[:, :HH]
    w0_ref[1] = q4[:, HH:]
    for kk, k in enumerate((1, 2, 3)):
        qk = p[:, k * HP:(k + 1) * HP] * dis
        qs_ref[kk, 0] = qk[:, :HH]
        qs_ref[kk, 1] = qk[:, HH:]
    dis_ref[...] = jnp.broadcast_to(dis, (BN, HP))
    dds_ref[...] = jnp.broadcast_to(dis * dis, (BN, HH))


def _mm1(x_p, ccat, degp):
    return pl.pallas_call(
        _mm1_body,
        grid=(NPAD // BN,),
        in_specs=[
            pl.BlockSpec((BN, D), lambda i: (i, 0)),
            pl.BlockSpec((D, 5 * HP), lambda i: (0, 0)),
            pl.BlockSpec((2, BN, 16), lambda i: (0, i, 0)),
        ],
        out_specs=[
            pl.BlockSpec((5, BN, HP), lambda i: (0, i, 0)),
            pl.BlockSpec((2, BN, HH), lambda i: (0, i, 0)),
            pl.BlockSpec((3, 2, BN, HH), lambda i: (0, 0, i, 0)),
            pl.BlockSpec((BN, HP), lambda i: (i, 0)),
            pl.BlockSpec((BN, HH), lambda i: (i, 0)),
        ],
        out_shape=[
            jax.ShapeDtypeStruct((5, NPAD, HP), jnp.float32),
            jax.ShapeDtypeStruct((2, NPAD, HH), jnp.float32),
            jax.ShapeDtypeStruct((3, 2, NPAD, HH), jnp.float32),
            jax.ShapeDtypeStruct((NPAD, HP), jnp.float32),
            jax.ShapeDtypeStruct((NPAD, HH), jnp.float32),
        ],
    )(x_p, ccat, degp)


def _final_body(p_ref, dis_ref, p0_ref, bc_ref, wf_ref, bf_ref, o_ref):
    srow = jnp.concatenate(
        [p_ref[0, 0] + p_ref[1, 0], p_ref[0, 1] + p_ref[1, 1]], axis=1)
    pre = p0_ref[...] - dis_ref[...] * srow + bc_ref[...]
    h = jnp.maximum(pre, 0.0)
    logits = jnp.dot(h, wf_ref[...], preferred_element_type=jnp.float32)
    logits = logits + bf_ref[...]
    m = jnp.max(logits, axis=1, keepdims=True)
    lse = jnp.log(jnp.sum(jnp.exp(logits - m), axis=1, keepdims=True)) + m
    o_ref[...] = logits - lse


def _final(part, dis2d, p0, bc_p, wf_p, bf_p):
    return pl.pallas_call(
        _final_body,
        grid=(NPAD // BN,),
        in_specs=[
            pl.BlockSpec((2, 2, BN, HH), lambda i: (0, 0, i, 0)),
            pl.BlockSpec((BN, HP), lambda i: (i, 0)),
            pl.BlockSpec((BN, HP), lambda i: (i, 0)),
            pl.BlockSpec((1, HP), lambda i: (0, 0)),
            pl.BlockSpec((HP, C), lambda i: (0, 0)),
            pl.BlockSpec((1, C), lambda i: (0, 0)),
        ],
        out_specs=pl.BlockSpec((BN, C), lambda i: (i, 0)),
        out_shape=jax.ShapeDtypeStruct((NPAD, C), jnp.float32),
    )(part, dis2d, p0, bc_p, wf_p, bf_p)


def kernel(x, edge_index, W_cheb, b_cheb, W_fc, b_fc):
    c0 = W_cheb[0] - W_cheb[2] + W_cheb[4]
    c1 = W_cheb[1] - 3.0 * W_cheb[3]
    c2 = 2.0 * W_cheb[2] - 8.0 * W_cheb[4]
    c3 = 4.0 * W_cheb[3]
    c4 = 8.0 * W_cheb[4]
    pad = [(0, 0), (0, HP - H)]
    ccat = jnp.concatenate(
        [jnp.pad(m, pad) for m in (c4, c3, c2, c1, c0)], axis=1)

    x_p = jnp.pad(x, [(0, NPAD - N), (0, 0)])
    row_p = jnp.pad(edge_index[0], (0, EPAD - edge_index.shape[1]),
                    constant_values=N).reshape(TOTCH, CE)
    col_p = jnp.pad(edge_index[1], (0, EPAD - edge_index.shape[1]),
                    constant_values=N).reshape(TOTCH, CE)

    bc_p = jnp.pad(b_cheb, (0, HP - H)).reshape(1, HP)
    wf_p = jnp.pad(W_fc, [(0, HP - H), (0, 0)])
    bf_p = b_fc.reshape(1, C)

    degp = _deg_kernel(row_p)
    qp5, w0, qs, dis2d, dds = _mm1(x_p, ccat, degp)
    part = _mega_kernel(w0, row_p, col_p, qs, dds)
    out = _final(part, dis2d, qp5[4], bc_p, wf_p, bf_p)
    return out[:N]
